# Initial kernel scaffold; baseline (speedup 1.0000x reference)
import functools

import jax
import jax.numpy as jnp
from jax import lax
from jax.experimental import pallas as pl
from jax.experimental.pallas import tpu as pltpu
from jax.experimental.pallas import tpu_sc as plsc

NN = 10000
EE = 160000
DIN = 256
DHALF = 128
BATCH = 128
NBATCH = EE // BATCH
NTILES = 16
RPT = NN // NTILES
MMBLK = 2000
DBLK = 1250


def _mm_body(x_ref, wl_ref, wr_ref, bl_ref, ycat_ref, yr_ref):
    xx = x_ref[...]
    yl = jnp.dot(xx, wl_ref[...], preferred_element_type=jnp.float32)
    ycat_ref[0] = yl[:, :DHALF]
    ycat_ref[1] = yl[:, DHALF:]
    yr_ref[...] = jnp.dot(xx, wr_ref[...], preferred_element_type=jnp.float32) + bl_ref[...]


def _matmul_stage(x, W_l1, W_r1, b_l1_2d):
    return pl.pallas_call(
        _mm_body,
        grid=(NN // MMBLK,),
        in_specs=[
            pl.BlockSpec((MMBLK, DIN), lambda i: (i, 0)),
            pl.BlockSpec((DIN, DIN), lambda i: (0, 0)),
            pl.BlockSpec((DIN, DIN), lambda i: (0, 0)),
            pl.BlockSpec((1, DIN), lambda i: (0, 0)),
        ],
        out_specs=[
            pl.BlockSpec((2, MMBLK, DHALF), lambda i: (0, i, 0)),
            pl.BlockSpec((MMBLK, DIN), lambda i: (i, 0)),
        ],
        out_shape=[
            jax.ShapeDtypeStruct((2, NN, DHALF), jnp.float32),
            jax.ShapeDtypeStruct((NN, DIN), jnp.float32),
        ],
    )(x, W_l1, W_r1, b_l1_2d)


def _agg_body(src_hbm, dst_hbm, ycat_hbm,
              acc0_out, acc1_out, cnt0_out, cnt1_out,
              src_idx, dst_idx, rows, ones, zacc, zcnt, acc_sh, cnt_sh, sem):
    c = lax.axis_index("c")
    s = lax.axis_index("s")
    row0 = s * RPT

    def _zacc_row(i, _):
        for jj in range(DHALF // 16):
            zacc[i, pl.ds(jj * 16, 16)] = jnp.zeros((16,), jnp.float32)
        return 0
    lax.fori_loop(0, RPT // 5, _zacc_row, 0)

    def _zcnt_row(i, _):
        zcnt[i, pl.ds(0, 16)] = jnp.zeros((16,), jnp.float32)
        return 0
    lax.fori_loop(0, RPT, _zcnt_row, 0)

    def _ones_row(i, _):
        ones[i, pl.ds(0, 16)] = jnp.ones((16,), jnp.float32)
        return 0
    lax.fori_loop(0, BATCH, _ones_row, 0)

    for k in range(5):
        pltpu.sync_copy(zacc, acc_sh.at[pl.ds(row0 + k * (RPT // 5), RPT // 5)])
    pltpu.sync_copy(zcnt, cnt_sh.at[pl.ds(row0, RPT)])
    plsc.subcore_barrier()

    nb = jnp.where(s < NBATCH - (NBATCH // NTILES) * NTILES,
                   NBATCH // NTILES + 1, NBATCH // NTILES)
    coff = c * NN

    def _batch(k, _):
        off = (s + k * NTILES) * BATCH
        pltpu.sync_copy(src_hbm.at[pl.ds(off, BATCH)], src_idx)
        pltpu.sync_copy(dst_hbm.at[pl.ds(off, BATCH)], dst_idx)
        for g in range(BATCH // 16):
            src_idx[pl.ds(g * 16, 16)] = src_idx[pl.ds(g * 16, 16)] + coff
        pltpu.async_copy(ycat_hbm.at[src_idx], rows, sem).wait()
        pltpu.sync_copy(rows, acc_sh.at[dst_idx], add=True)

        @pl.when(s % 2 == c)
        def _():
            pltpu.sync_copy(ones, cnt_sh.at[dst_idx], add=True)
        return 0

    lax.fori_loop(0, nb, _batch, 0)
    plsc.subcore_barrier()

    @pl.when(c == 0)
    def _():
        pltpu.sync_copy(acc_sh.at[pl.ds(row0, RPT)], acc0_out.at[pl.ds(row0, RPT)])
        pltpu.sync_copy(cnt_sh.at[pl.ds(row0, RPT)], cnt0_out.at[pl.ds(row0, RPT)])

    @pl.when(c == 1)
    def _():
        pltpu.sync_copy(acc_sh.at[pl.ds(row0, RPT)], acc1_out.at[pl.ds(row0, RPT)])
        pltpu.sync_copy(cnt_sh.at[pl.ds(row0, RPT)], cnt1_out.at[pl.ds(row0, RPT)])


def _aggregate(src, dst, y_cat_flat):
    f32 = jnp.float32
    kern = functools.partial(
        pl.kernel,
        mesh=plsc.VectorSubcoreMesh(core_axis_name="c", subcore_axis_name="s"),
        out_type=[
            jax.ShapeDtypeStruct((NN, DHALF), f32),
            jax.ShapeDtypeStruct((NN, DHALF), f32),
            jax.ShapeDtypeStruct((NN, 16), f32),
            jax.ShapeDtypeStruct((NN, 16), f32),
        ],
        scratch_types=[
            pltpu.VMEM((BATCH,), jnp.int32),
            pltpu.VMEM((BATCH,), jnp.int32),
            pltpu.VMEM((BATCH, DHALF), f32),
            pltpu.VMEM((BATCH, 16), f32),
            pltpu.VMEM((RPT // 5, DHALF), f32),
            pltpu.VMEM((RPT, 16), f32),
            pltpu.VMEM_SHARED((NN, DHALF), f32),
            pltpu.VMEM_SHARED((NN, 16), f32),
            pltpu.SemaphoreType.DMA,
        ],
    )
    return kern(_agg_body)(src, dst, y_cat_flat)


def _w_body(src_hbm, dst_hbm, cnt_hbm, w0_out, w1_out,
            cnt_vmem, src_idx, dst_slab, stage, zw, w_sh, sem):
    c = lax.axis_index("c")
    s = lax.axis_index("s")
    wid = s * 2 + c
    row0 = s * RPT

    pltpu.sync_copy(cnt_hbm, cnt_vmem)

    def _zw_row(i, _):
        zw[i, pl.ds(0, 16)] = jnp.zeros((16,), jnp.float32)
        return 0
    lax.fori_loop(0, RPT, _zw_row, 0)

    def _zstage_row(i, _):
        stage[i, pl.ds(0, 16)] = jnp.zeros((16,), jnp.float32)
        return 0
    lax.fori_loop(0, BATCH, _zstage_row, 0)

    pltpu.sync_copy(zw, w_sh.at[pl.ds(row0, RPT)])
    plsc.subcore_barrier()

    nw = NTILES * 2
    nb = jnp.where(wid < NBATCH - (NBATCH // nw) * nw,
                   NBATCH // nw + 1, NBATCH // nw)
    col0 = jnp.zeros((16,), jnp.int32)

    def _batch(k, _):
        off = (wid + k * nw) * BATCH
        pltpu.sync_copy(src_hbm.at[pl.ds(off, BATCH)], src_idx)
        pltpu.sync_copy(dst_hbm.at[pl.ds(off, BATCH)], dst_slab)
        for g in range(BATCH // 16):
            dstv = dst_slab[pl.ds(g * 16, 16)]
            cv = plsc.load_gather(cnt_vmem, [dstv])
            r = 1.0 / jnp.maximum(cv, 1.0)
            rowv = lax.iota(jnp.int32, 16) + g * 16
            plsc.store_scatter(stage, [rowv, col0], r)
        pltpu.sync_copy(stage, w_sh.at[src_idx], add=True)
        return 0

    lax.fori_loop(0, nb, _batch, 0)
    plsc.subcore_barrier()

    @pl.when(c == 0)
    def _():
        pltpu.sync_copy(w_sh.at[pl.ds(row0, RPT)], w0_out.at[pl.ds(row0, RPT)])

    @pl.when(c == 1)
    def _():
        pltpu.sync_copy(w_sh.at[pl.ds(row0, RPT)], w1_out.at[pl.ds(row0, RPT)])


def _edge_weights(src, dst, cnt):
    f32 = jnp.float32
    kern = functools.partial(
        pl.kernel,
        mesh=plsc.VectorSubcoreMesh(core_axis_name="c", subcore_axis_name="s"),
        out_type=[
            jax.ShapeDtypeStruct((NN, 16), f32),
            jax.ShapeDtypeStruct((NN, 16), f32),
        ],
        scratch_types=[
            pltpu.VMEM((NN,), f32),
            pltpu.VMEM((BATCH,), jnp.int32),
            pltpu.VMEM((BATCH,), jnp.int32),
            pltpu.VMEM((BATCH, 16), f32),
            pltpu.VMEM((RPT, 16), f32),
            pltpu.VMEM_SHARED((NN, 16), f32),
            pltpu.SemaphoreType.DMA,
        ],
    )
    return kern(_w_body)(src, dst, cnt)


def _tail_body(acc0_ref, acc1_ref, yr_ref, cnt_ref, w_ref,
               gamma_ref, beta_ref, wl2_ref, bl2_ref, wr2_ref,
               wsk_ref, bsk_ref, wc1_ref, bc1_ref, wc2_ref, bc2_ref,
               out_ref, sum_ref, sq_ref, s1_ref, m1_ref):
    p = pl.program_id(0)
    i = pl.program_id(1)

    @pl.when((p == 0) & (i == 0))
    def _():
        sum_ref[...] = jnp.zeros_like(sum_ref)
        sq_ref[...] = jnp.zeros_like(sq_ref)

    @pl.when((p == 1) & (i == 0))
    def _():
        s1_ref[...] = jnp.zeros_like(s1_ref)
        m1_ref[...] = jnp.zeros_like(m1_ref)

    acc = jnp.concatenate([acc0_ref[...], acc1_ref[...]], axis=1)
    z = acc / jnp.maximum(cnt_ref[...], 1.0) + yr_ref[...]

    @pl.when(p == 0)
    def _():
        sum_ref[...] += jnp.sum(z, axis=0, keepdims=True)
        sq_ref[...] += jnp.sum(z * z, axis=0, keepdims=True)

    @pl.when(p == 1)
    def _():
        mu = sum_ref[...] * (1.0 / NN)
        var = sq_ref[...] * (1.0 / NN) - mu * mu
        inv = lax.rsqrt(var + 1e-5)
        h = jnp.maximum((z - mu) * (inv * gamma_ref[...]) + beta_ref[...], 0.0)
        s1_ref[...] += jnp.sum(h * w_ref[...], axis=0, keepdims=True)
        m1_ref[...] += jnp.sum(h, axis=0, keepdims=True)

    @pl.when((p == 1) & (i == NN // DBLK - 1))
    def _():
        s1 = s1_ref[...] * (1.0 / NN)
        m1 = m1_ref[...] * (1.0 / NN)
        mh2 = (jnp.dot(s1, wl2_ref[...], preferred_element_type=jnp.float32)
               + bl2_ref[...]
               + jnp.dot(m1, wr2_ref[...], preferred_element_type=jnp.float32))
        pooled = jnp.dot(mh2, wsk_ref[...], preferred_element_type=jnp.float32) + bsk_ref[...]
        hid = jnp.maximum(
            jnp.dot(pooled, wc1_ref[...], preferred_element_type=jnp.float32) + bc1_ref[...], 0.0)
        out_ref[...] = jnp.dot(hid, wc2_ref[...], preferred_element_type=jnp.float32) + bc2_ref[...]


def _finalize(acc0, acc1, yr, cnt2, w2, gamma2, beta2, W_l2, b_l2_2, W_r2,
              W_sk, b_sk_2, W_c1, b_c1_2, W_c2, b_c2_2):
    def full(a):
        n = len(a.shape)
        return pl.BlockSpec(a.shape, lambda p, i, _n=n: (0,) * _n)
    return pl.pallas_call(
        _tail_body,
        grid=(2, NN // DBLK),
        in_specs=[
            pl.BlockSpec((DBLK, DHALF), lambda p, i: (i, 0)),
            pl.BlockSpec((DBLK, DHALF), lambda p, i: (i, 0)),
            pl.BlockSpec((DBLK, DIN), lambda p, i: (i, 0)),
            pl.BlockSpec((DBLK, 1), lambda p, i: (i, 0)),
            pl.BlockSpec((DBLK, 1), lambda p, i: (i, 0)),
            full(gamma2), full(beta2), full(W_l2), full(b_l2_2), full(W_r2),
            full(W_sk), full(b_sk_2), full(W_c1), full(b_c1_2), full(W_c2), full(b_c2_2),
        ],
        out_specs=pl.BlockSpec((1, 2), lambda p, i: (0, 0)),
        out_shape=jax.ShapeDtypeStruct((1, 2), jnp.float32),
        scratch_shapes=[
            pltpu.VMEM((1, DIN), jnp.float32),
            pltpu.VMEM((1, DIN), jnp.float32),
            pltpu.VMEM((1, DIN), jnp.float32),
            pltpu.VMEM((1, DIN), jnp.float32),
        ],
    )(acc0, acc1, yr, cnt2, w2, gamma2, beta2, W_l2, b_l2_2, W_r2,
      W_sk, b_sk_2, W_c1, b_c1_2, W_c2, b_c2_2)


def kernel(x, edge_index, W_l1, b_l1, W_r1, gamma, beta, W_l2, b_l2, W_r2,
           W_sk, b_sk, W_c1, b_c1, W_c2, b_c2):
    src = edge_index[0]
    dst = edge_index[1]

    y_cat, y_r = _matmul_stage(x, W_l1, W_r1, b_l1.reshape(1, -1))
    acc0, acc1, cnt0, cnt1 = _aggregate(src, dst, y_cat.reshape(2 * NN, DHALF))
    cnt = cnt0[:, 0] + cnt1[:, 0]
    w0, w1 = _edge_weights(src, dst, cnt)
    w = w0[:, 0] + w1[:, 0]

    return _finalize(
        acc0, acc1, y_r, cnt[:, None], w[:, None],
        gamma.reshape(1, -1), beta.reshape(1, -1), W_l2, b_l2.reshape(1, -1),
        W_r2, W_sk, b_sk.reshape(1, -1), W_c1, b_c1.reshape(1, -1),
        W_c2, b_c2.reshape(1, -1))


# speedup vs baseline: 5.6995x; 5.6995x over previous
import functools

import jax
import jax.numpy as jnp
from jax import lax
from jax.experimental import pallas as pl
from jax.experimental.pallas import tpu as pltpu
from jax.experimental.pallas import tpu_sc as plsc

_PHI = jax.lax.Precision.HIGHEST

NN = 10000
EE = 160000
DIN = 256
DHALF = 128
BATCH = 128
NBATCH = EE // BATCH
NTILES = 16
RPT = NN // NTILES
MMBLK = 2000
DBLK = 1000


def _mm_body(x_ref, wl_ref, wr_ref, bl_ref, ycat_ref, yr_ref):
    xx = x_ref[...]
    yl = jnp.dot(xx, wl_ref[...], preferred_element_type=jnp.float32, precision=_PHI)
    ycat_ref[0] = yl[:, :DHALF]
    ycat_ref[1] = yl[:, DHALF:]
    yr_ref[...] = jnp.dot(xx, wr_ref[...], preferred_element_type=jnp.float32, precision=_PHI) + bl_ref[...]


def _matmul_stage(x, W_l1, W_r1, b_l1_2d):
    return pl.pallas_call(
        _mm_body,
        grid=(NN // MMBLK,),
        in_specs=[
            pl.BlockSpec((MMBLK, DIN), lambda i: (i, 0)),
            pl.BlockSpec((DIN, DIN), lambda i: (0, 0)),
            pl.BlockSpec((DIN, DIN), lambda i: (0, 0)),
            pl.BlockSpec((1, DIN), lambda i: (0, 0)),
        ],
        out_specs=[
            pl.BlockSpec((2, MMBLK, DHALF), lambda i: (0, i, 0)),
            pl.BlockSpec((MMBLK, DIN), lambda i: (i, 0)),
        ],
        out_shape=[
            jax.ShapeDtypeStruct((2, NN, DHALF), jnp.float32),
            jax.ShapeDtypeStruct((NN, DIN), jnp.float32),
        ],
    )(x, W_l1, W_r1, b_l1_2d)


def _zero_rows(buf, nrows, width):
    def _row(i, _):
        for jj in range(width // 16):
            buf[i, pl.ds(jj * 16, 16)] = jnp.zeros((16,), jnp.float32)
        return 0
    lax.fori_loop(0, nrows, _row, 0)


def _zero_spmem(sh, zsrc, s, width):
    nzc = NN // 128
    nz = jnp.where(s < nzc - (nzc // NTILES) * NTILES,
                   nzc // NTILES + 1, nzc // NTILES)

    def _zchunk(k, _):
        m = (s + k * NTILES) * 128
        pltpu.sync_copy(zsrc, sh.at[pl.ds(m, 128)])
        return 0
    lax.fori_loop(0, nz, _zchunk, 0)

    @pl.when(s == NTILES - 1)
    def _():
        pltpu.sync_copy(zsrc.at[pl.ds(0, 16)], sh.at[pl.ds(nzc * 128, 16)])


def _agg_body(src_hbm, dst_hbm, ycat_hbm, acc0_out, acc1_out,
              src_idx, dst_idx, rows, acc_sh, sem):
    c = lax.axis_index("c")
    s = lax.axis_index("s")
    row0 = s * RPT

    _zero_rows(rows, BATCH, DHALF)
    _zero_spmem(acc_sh, rows, s, DHALF)
    plsc.subcore_barrier()

    nb = jnp.where(s < NBATCH - (NBATCH // NTILES) * NTILES,
                   NBATCH // NTILES + 1, NBATCH // NTILES)
    coff = c * NN

    def _batch(k, _):
        off = (s + k * NTILES) * BATCH
        pltpu.sync_copy(src_hbm.at[pl.ds(off, BATCH)], src_idx)
        pltpu.sync_copy(dst_hbm.at[pl.ds(off, BATCH)], dst_idx)
        for g in range(BATCH // 16):
            src_idx[pl.ds(g * 16, 16)] = src_idx[pl.ds(g * 16, 16)] + coff
        pltpu.async_copy(ycat_hbm.at[src_idx], rows, sem).wait()
        pltpu.sync_copy(rows, acc_sh.at[dst_idx], add=True)
        return 0

    lax.fori_loop(0, nb, _batch, 0)
    plsc.subcore_barrier()

    @pl.when(c == 0)
    def _():
        pltpu.sync_copy(acc_sh.at[pl.ds(row0, RPT)], acc0_out.at[s])

    @pl.when(c == 1)
    def _():
        pltpu.sync_copy(acc_sh.at[pl.ds(row0, RPT)], acc1_out.at[s])


def _aggregate(src, dst, y_cat_flat):
    f32 = jnp.float32
    kern = functools.partial(
        pl.kernel,
        mesh=plsc.VectorSubcoreMesh(core_axis_name="c", subcore_axis_name="s"),
        compiler_params=pltpu.CompilerParams(needs_layout_passes=False),
        out_type=[
            jax.ShapeDtypeStruct((NTILES, RPT, DHALF), f32),
            jax.ShapeDtypeStruct((NTILES, RPT, DHALF), f32),
        ],
        scratch_types=[
            pltpu.VMEM((BATCH,), jnp.int32),
            pltpu.VMEM((BATCH,), jnp.int32),
            pltpu.VMEM((BATCH, DHALF), f32),
            pltpu.VMEM_SHARED((NN, DHALF), f32),
            pltpu.SemaphoreType.DMA,
        ],
    )
    return kern(_agg_body)(src, dst, y_cat_flat)


def _cnt_body(dst_hbm, cnt0_out, cnt1_out, dst_idx, ones, cnt_sh, sem):
    c = lax.axis_index("c")
    s = lax.axis_index("s")
    row0 = s * RPT

    _zero_rows(ones, BATCH, DHALF)
    _zero_spmem(cnt_sh, ones, s, DHALF)

    def _ones_row(i, _):
        for jj in range(DHALF // 16):
            ones[i, pl.ds(jj * 16, 16)] = jnp.ones((16,), jnp.float32)
        return 0
    lax.fori_loop(0, BATCH, _ones_row, 0)
    plsc.subcore_barrier()

    nhalf = NBATCH // 2
    nb = jnp.where(s < nhalf - (nhalf // NTILES) * NTILES,
                   nhalf // NTILES + 1, nhalf // NTILES)

    def _batch(k, _):
        off = (2 * (s + k * NTILES) + c) * BATCH
        pltpu.sync_copy(dst_hbm.at[pl.ds(off, BATCH)], dst_idx)
        pltpu.sync_copy(ones, cnt_sh.at[dst_idx], add=True)
        return 0

    lax.fori_loop(0, nb, _batch, 0)
    plsc.subcore_barrier()

    @pl.when(c == 0)
    def _():
        pltpu.sync_copy(cnt_sh.at[pl.ds(row0, RPT)], cnt0_out.at[s])

    @pl.when(c == 1)
    def _():
        pltpu.sync_copy(cnt_sh.at[pl.ds(row0, RPT)], cnt1_out.at[s])


def _degree_counts(dst):
    f32 = jnp.float32
    kern = functools.partial(
        pl.kernel,
        mesh=plsc.VectorSubcoreMesh(core_axis_name="c", subcore_axis_name="s"),
        compiler_params=pltpu.CompilerParams(needs_layout_passes=False),
        out_type=[
            jax.ShapeDtypeStruct((NTILES, RPT, DHALF), f32),
            jax.ShapeDtypeStruct((NTILES, RPT, DHALF), f32),
        ],
        scratch_types=[
            pltpu.VMEM((BATCH,), jnp.int32),
            pltpu.VMEM((BATCH, DHALF), f32),
            pltpu.VMEM_SHARED((NN, DHALF), f32),
            pltpu.SemaphoreType.DMA,
        ],
    )
    return kern(_cnt_body)(dst)


def _w_body(src_hbm, dst_hbm, cnt_hbm, w0_out, w1_out,
            cnt_vmem, src_idx, dst_slab, stage, w_sh, sem):
    c = lax.axis_index("c")
    s = lax.axis_index("s")
    wid = s * 2 + c
    row0 = s * RPT

    pltpu.sync_copy(cnt_hbm, cnt_vmem)

    _zero_rows(stage, BATCH, DHALF)
    _zero_spmem(w_sh, stage, s, DHALF)
    plsc.subcore_barrier()

    nw = NTILES * 2
    nb = jnp.where(wid < NBATCH - (NBATCH // nw) * nw,
                   NBATCH // nw + 1, NBATCH // nw)
    col0 = jnp.zeros((16,), jnp.int32)

    def _batch(k, _):
        off = (wid + k * nw) * BATCH
        pltpu.sync_copy(src_hbm.at[pl.ds(off, BATCH)], src_idx)
        pltpu.sync_copy(dst_hbm.at[pl.ds(off, BATCH)], dst_slab)
        for g in range(BATCH // 16):
            dstv = dst_slab[pl.ds(g * 16, 16)]
            cv = plsc.load_gather(cnt_vmem, [dstv])
            r = 1.0 / jnp.maximum(cv, 1.0)
            rowv = lax.iota(jnp.int32, 16) + g * 16
            plsc.store_scatter(stage, [rowv, col0], r)
        pltpu.sync_copy(stage, w_sh.at[src_idx], add=True)
        return 0

    lax.fori_loop(0, nb, _batch, 0)
    plsc.subcore_barrier()

    @pl.when(c == 0)
    def _():
        pltpu.sync_copy(w_sh.at[pl.ds(row0, RPT)], w0_out.at[s])

    @pl.when(c == 1)
    def _():
        pltpu.sync_copy(w_sh.at[pl.ds(row0, RPT)], w1_out.at[s])


def _edge_weights(src, dst, cnt):
    f32 = jnp.float32
    kern = functools.partial(
        pl.kernel,
        mesh=plsc.VectorSubcoreMesh(core_axis_name="c", subcore_axis_name="s"),
        compiler_params=pltpu.CompilerParams(needs_layout_passes=False),
        out_type=[
            jax.ShapeDtypeStruct((NTILES, RPT, DHALF), f32),
            jax.ShapeDtypeStruct((NTILES, RPT, DHALF), f32),
        ],
        scratch_types=[
            pltpu.VMEM((NN,), f32),
            pltpu.VMEM((BATCH,), jnp.int32),
            pltpu.VMEM((BATCH,), jnp.int32),
            pltpu.VMEM((BATCH, DHALF), f32),
            pltpu.VMEM_SHARED((NN, DHALF), f32),
            pltpu.SemaphoreType.DMA,
        ],
    )
    return kern(_w_body)(src, dst, cnt)


def _tail_body(acc0_ref, acc1_ref, yr_ref, cnt_ref, w_ref,
               gamma_ref, beta_ref, wl2_ref, bl2_ref, wr2_ref,
               wsk_ref, bsk_ref, wc1_ref, bc1_ref, wc2_ref, bc2_ref,
               out_ref, sum_ref, sq_ref, s1_ref, m1_ref):
    p = pl.program_id(0)
    i = pl.program_id(1)

    @pl.when((p == 0) & (i == 0))
    def _():
        sum_ref[...] = jnp.zeros_like(sum_ref)

    @pl.when((p == 1) & (i == 0))
    def _():
        sq_ref[...] = jnp.zeros_like(sq_ref)

    @pl.when((p == 2) & (i == 0))
    def _():
        s1_ref[...] = jnp.zeros_like(s1_ref)
        m1_ref[...] = jnp.zeros_like(m1_ref)

    acc = jnp.concatenate([acc0_ref[...], acc1_ref[...]], axis=1)
    z = acc / jnp.maximum(cnt_ref[...], 1.0) + yr_ref[...]

    @pl.when(p == 0)
    def _():
        sum_ref[...] += jnp.sum(z, axis=0, keepdims=True)

    @pl.when(p == 1)
    def _():
        zc = z - sum_ref[...] * (1.0 / NN)
        sq_ref[...] += jnp.sum(zc * zc, axis=0, keepdims=True)

    @pl.when(p == 2)
    def _():
        mu = sum_ref[...] * (1.0 / NN)
        var = sq_ref[...] * (1.0 / NN)
        inv = 1.0 / jnp.sqrt(var + 1e-5)
        h = jnp.maximum((z - mu) * (inv * gamma_ref[...]) + beta_ref[...], 0.0)
        s1_ref[...] += jnp.sum(h * w_ref[...], axis=0, keepdims=True)
        m1_ref[...] += jnp.sum(h, axis=0, keepdims=True)

    @pl.when((p == 2) & (i == NN // DBLK - 1))
    def _():
        s1 = s1_ref[...] * (1.0 / NN)
        m1 = m1_ref[...] * (1.0 / NN)
        mh2 = (jnp.dot(s1, wl2_ref[...], preferred_element_type=jnp.float32, precision=_PHI)
               + bl2_ref[...]
               + jnp.dot(m1, wr2_ref[...], preferred_element_type=jnp.float32, precision=_PHI))
        pooled = jnp.dot(mh2, wsk_ref[...], preferred_element_type=jnp.float32, precision=_PHI) + bsk_ref[...]
        hid = jnp.maximum(
            jnp.dot(pooled, wc1_ref[...], preferred_element_type=jnp.float32, precision=_PHI) + bc1_ref[...], 0.0)
        out_ref[...] = jnp.dot(hid, wc2_ref[...], preferred_element_type=jnp.float32, precision=_PHI) + bc2_ref[...]


def _finalize(acc0, acc1, yr, cnt2, w2, gamma2, beta2, W_l2, b_l2_2, W_r2,
              W_sk, b_sk_2, W_c1, b_c1_2, W_c2, b_c2_2):
    def full(a):
        n = len(a.shape)
        return pl.BlockSpec(a.shape, lambda p, i, _n=n: (0,) * _n)
    return pl.pallas_call(
        _tail_body,
        grid=(3, NN // DBLK),
        in_specs=[
            pl.BlockSpec((DBLK, DHALF), lambda p, i: (i, 0)),
            pl.BlockSpec((DBLK, DHALF), lambda p, i: (i, 0)),
            pl.BlockSpec((DBLK, DIN), lambda p, i: (i, 0)),
            pl.BlockSpec((DBLK, 1), lambda p, i: (i, 0)),
            pl.BlockSpec((DBLK, 1), lambda p, i: (i, 0)),
            full(gamma2), full(beta2), full(W_l2), full(b_l2_2), full(W_r2),
            full(W_sk), full(b_sk_2), full(W_c1), full(b_c1_2), full(W_c2), full(b_c2_2),
        ],
        out_specs=pl.BlockSpec((1, 2), lambda p, i: (0, 0)),
        out_shape=jax.ShapeDtypeStruct((1, 2), jnp.float32),
        scratch_shapes=[
            pltpu.VMEM((1, DIN), jnp.float32),
            pltpu.VMEM((1, DIN), jnp.float32),
            pltpu.VMEM((1, DIN), jnp.float32),
            pltpu.VMEM((1, DIN), jnp.float32),
        ],
    )(acc0, acc1, yr, cnt2, w2, gamma2, beta2, W_l2, b_l2_2, W_r2,
      W_sk, b_sk_2, W_c1, b_c1_2, W_c2, b_c2_2)


def kernel(x, edge_index, W_l1, b_l1, W_r1, gamma, beta, W_l2, b_l2, W_r2,
           W_sk, b_sk, W_c1, b_c1, W_c2, b_c2):
    src = edge_index[0]
    dst = edge_index[1]

    y_cat, y_r = _matmul_stage(x, W_l1, W_r1, b_l1.reshape(1, -1))
    acc0, acc1 = _aggregate(src, dst, y_cat.reshape(2 * NN, DHALF))
    acc0 = acc0.reshape(NN, DHALF)
    acc1 = acc1.reshape(NN, DHALF)
    cnt0, cnt1 = _degree_counts(dst)
    cnt = cnt0.reshape(NN, DHALF)[:, 0] + cnt1.reshape(NN, DHALF)[:, 0]
    w0, w1 = _edge_weights(src, dst, cnt)
    w = w0.reshape(NN, DHALF)[:, 0] + w1.reshape(NN, DHALF)[:, 0]

    return _finalize(
        acc0, acc1, y_r, cnt[:, None], w[:, None],
        gamma.reshape(1, -1), beta.reshape(1, -1), W_l2, b_l2.reshape(1, -1),
        W_r2, W_sk, b_sk.reshape(1, -1), W_c1, b_c1.reshape(1, -1),
        W_c2, b_c2.reshape(1, -1))


# speedup vs baseline: 7.1647x; 1.2571x over previous
import functools

import jax
import jax.numpy as jnp
from jax import lax
from jax.experimental import pallas as pl
from jax.experimental.pallas import tpu as pltpu
from jax.experimental.pallas import tpu_sc as plsc

_PHI = jax.lax.Precision.HIGHEST

NN = 10000
EE = 160000
DIN = 256
DHALF = 128
BATCH = 128
NBATCH = EE // BATCH
NTILES = 16
RPT = NN // NTILES
MMBLK = 2000
DBLK = 1000


def _mm_body(x_ref, wl_ref, wr_ref, bl_ref, ycat_ref, yr_ref):
    xx = x_ref[...]
    yl = jnp.dot(xx, wl_ref[...], preferred_element_type=jnp.float32, precision=_PHI)
    ycat_ref[0] = yl[:, :DHALF]
    ycat_ref[1] = yl[:, DHALF:]
    yr_ref[...] = jnp.dot(xx, wr_ref[...], preferred_element_type=jnp.float32, precision=_PHI) + bl_ref[...]


def _matmul_stage(x, W_l1, W_r1, b_l1_2d):
    return pl.pallas_call(
        _mm_body,
        grid=(NN // MMBLK,),
        in_specs=[
            pl.BlockSpec((MMBLK, DIN), lambda i: (i, 0)),
            pl.BlockSpec((DIN, DIN), lambda i: (0, 0)),
            pl.BlockSpec((DIN, DIN), lambda i: (0, 0)),
            pl.BlockSpec((1, DIN), lambda i: (0, 0)),
        ],
        out_specs=[
            pl.BlockSpec((2, MMBLK, DHALF), lambda i: (0, i, 0)),
            pl.BlockSpec((MMBLK, DIN), lambda i: (i, 0)),
        ],
        out_shape=[
            jax.ShapeDtypeStruct((2, NN, DHALF), jnp.float32),
            jax.ShapeDtypeStruct((NN, DIN), jnp.float32),
        ],
    )(x, W_l1, W_r1, b_l1_2d)


def _zero_rows(buf, nrows, width):
    def _row(i, _):
        for jj in range(width // 16):
            buf[i, pl.ds(jj * 16, 16)] = jnp.zeros((16,), jnp.float32)
        return 0
    lax.fori_loop(0, nrows, _row, 0)


def _zero_spmem(sh, zsrc, s, width):
    nzc = NN // 128
    nz = jnp.where(s < nzc - (nzc // NTILES) * NTILES,
                   nzc // NTILES + 1, nzc // NTILES)

    def _zchunk(k, _):
        m = (s + k * NTILES) * 128
        pltpu.sync_copy(zsrc, sh.at[pl.ds(m, 128)])
        return 0
    lax.fori_loop(0, nz, _zchunk, 0)

    @pl.when(s == NTILES - 1)
    def _():
        pltpu.sync_copy(zsrc.at[pl.ds(0, 16)], sh.at[pl.ds(nzc * 128, 16)])


def _agg_body(src_hbm, dst_hbm, ycat_hbm, acc0_out, acc1_out,
              src_a, dst_a, rows_a, src_b, dst_b, rows_b, acc_sh,
              sem_a, sem_b):
    c = lax.axis_index("c")
    s = lax.axis_index("s")
    row0 = s * RPT

    _zero_rows(rows_a, BATCH, DHALF)
    _zero_spmem(acc_sh, rows_a, s, DHALF)
    plsc.subcore_barrier()

    nb = jnp.where(s < NBATCH - (NBATCH // NTILES) * NTILES,
                   NBATCH // NTILES + 1, NBATCH // NTILES)
    coff = c * NN

    def _start(j, src_v, dst_v, rows_v, sem):
        off = (s + j * NTILES) * BATCH
        pltpu.sync_copy(src_hbm.at[pl.ds(off, BATCH)], src_v)
        pltpu.sync_copy(dst_hbm.at[pl.ds(off, BATCH)], dst_v)
        for g in range(BATCH // 16):
            src_v[pl.ds(g * 16, 16)] = src_v[pl.ds(g * 16, 16)] + coff
        return pltpu.async_copy(ycat_hbm.at[src_v], rows_v, sem)

    def _drain(src_v, dst_v, rows_v, sem):
        pltpu.make_async_copy(ycat_hbm.at[src_v], rows_v, sem).wait()
        pltpu.sync_copy(rows_v, acc_sh.at[dst_v], add=True)

    _start(0, src_a, dst_a, rows_a, sem_a)
    npairs = (nb + 1) // 2

    def _pair(m, _):
        j1 = 2 * m + 1

        @pl.when(j1 < nb)
        def _():
            _start(j1, src_b, dst_b, rows_b, sem_b)
        _drain(src_a, dst_a, rows_a, sem_a)

        @pl.when(j1 < nb)
        def _():
            @pl.when(j1 + 1 < nb)
            def _():
                _start(j1 + 1, src_a, dst_a, rows_a, sem_a)
            _drain(src_b, dst_b, rows_b, sem_b)
        return 0

    lax.fori_loop(0, npairs, _pair, 0)
    plsc.subcore_barrier()

    @pl.when(c == 0)
    def _():
        pltpu.sync_copy(acc_sh.at[pl.ds(row0, RPT)], acc0_out.at[s])

    @pl.when(c == 1)
    def _():
        pltpu.sync_copy(acc_sh.at[pl.ds(row0, RPT)], acc1_out.at[s])


def _aggregate(src, dst, y_cat_flat):
    f32 = jnp.float32
    kern = functools.partial(
        pl.kernel,
        mesh=plsc.VectorSubcoreMesh(core_axis_name="c", subcore_axis_name="s"),
        compiler_params=pltpu.CompilerParams(needs_layout_passes=False),
        out_type=[
            jax.ShapeDtypeStruct((NTILES, RPT, DHALF), f32),
            jax.ShapeDtypeStruct((NTILES, RPT, DHALF), f32),
        ],
        scratch_types=[
            pltpu.VMEM((BATCH,), jnp.int32),
            pltpu.VMEM((BATCH,), jnp.int32),
            pltpu.VMEM((BATCH, DHALF), f32),
            pltpu.VMEM((BATCH,), jnp.int32),
            pltpu.VMEM((BATCH,), jnp.int32),
            pltpu.VMEM((BATCH, DHALF), f32),
            pltpu.VMEM_SHARED((NN, DHALF), f32),
            pltpu.SemaphoreType.DMA,
            pltpu.SemaphoreType.DMA,
        ],
    )
    return kern(_agg_body)(src, dst, y_cat_flat)


def _cnt_body(dst_hbm, cnt0_out, cnt1_out, dst_idx, ones, cnt_sh, sem):
    c = lax.axis_index("c")
    s = lax.axis_index("s")
    row0 = s * RPT

    _zero_rows(ones, BATCH, DHALF)
    _zero_spmem(cnt_sh, ones, s, DHALF)

    def _ones_row(i, _):
        for jj in range(DHALF // 16):
            ones[i, pl.ds(jj * 16, 16)] = jnp.ones((16,), jnp.float32)
        return 0
    lax.fori_loop(0, BATCH, _ones_row, 0)
    plsc.subcore_barrier()

    nhalf = NBATCH // 2
    nb = jnp.where(s < nhalf - (nhalf // NTILES) * NTILES,
                   nhalf // NTILES + 1, nhalf // NTILES)

    def _batch(k, _):
        off = (2 * (s + k * NTILES) + c) * BATCH
        pltpu.sync_copy(dst_hbm.at[pl.ds(off, BATCH)], dst_idx)
        pltpu.sync_copy(ones, cnt_sh.at[dst_idx], add=True)
        return 0

    lax.fori_loop(0, nb, _batch, 0)
    plsc.subcore_barrier()

    @pl.when(c == 0)
    def _():
        pltpu.sync_copy(cnt_sh.at[pl.ds(row0, RPT)], cnt0_out.at[s])

    @pl.when(c == 1)
    def _():
        pltpu.sync_copy(cnt_sh.at[pl.ds(row0, RPT)], cnt1_out.at[s])


def _degree_counts(dst):
    f32 = jnp.float32
    kern = functools.partial(
        pl.kernel,
        mesh=plsc.VectorSubcoreMesh(core_axis_name="c", subcore_axis_name="s"),
        compiler_params=pltpu.CompilerParams(needs_layout_passes=False),
        out_type=[
            jax.ShapeDtypeStruct((NTILES, RPT, DHALF), f32),
            jax.ShapeDtypeStruct((NTILES, RPT, DHALF), f32),
        ],
        scratch_types=[
            pltpu.VMEM((BATCH,), jnp.int32),
            pltpu.VMEM((BATCH, DHALF), f32),
            pltpu.VMEM_SHARED((NN, DHALF), f32),
            pltpu.SemaphoreType.DMA,
        ],
    )
    return kern(_cnt_body)(dst)


def _w_body(src_hbm, dst_hbm, cnt_hbm, w0_out, w1_out,
            cnt_vmem, src_idx, dst_slab, stage, w_sh, sem):
    c = lax.axis_index("c")
    s = lax.axis_index("s")
    wid = s * 2 + c
    row0 = s * RPT

    pltpu.sync_copy(cnt_hbm, cnt_vmem)

    _zero_rows(stage, BATCH, DHALF)
    _zero_spmem(w_sh, stage, s, DHALF)
    plsc.subcore_barrier()

    nw = NTILES * 2
    nb = jnp.where(wid < NBATCH - (NBATCH // nw) * nw,
                   NBATCH // nw + 1, NBATCH // nw)
    col0 = jnp.zeros((16,), jnp.int32)

    def _batch(k, _):
        off = (wid + k * nw) * BATCH
        pltpu.sync_copy(src_hbm.at[pl.ds(off, BATCH)], src_idx)
        pltpu.sync_copy(dst_hbm.at[pl.ds(off, BATCH)], dst_slab)
        for g in range(BATCH // 16):
            dstv = dst_slab[pl.ds(g * 16, 16)]
            cv = plsc.load_gather(cnt_vmem, [dstv])
            r = 1.0 / jnp.maximum(cv, 1.0)
            rowv = lax.iota(jnp.int32, 16) + g * 16
            plsc.store_scatter(stage, [rowv, col0], r)
        pltpu.sync_copy(stage, w_sh.at[src_idx], add=True)
        return 0

    lax.fori_loop(0, nb, _batch, 0)
    plsc.subcore_barrier()

    @pl.when(c == 0)
    def _():
        pltpu.sync_copy(w_sh.at[pl.ds(row0, RPT)], w0_out.at[s])

    @pl.when(c == 1)
    def _():
        pltpu.sync_copy(w_sh.at[pl.ds(row0, RPT)], w1_out.at[s])


def _edge_weights(src, dst, cnt):
    f32 = jnp.float32
    kern = functools.partial(
        pl.kernel,
        mesh=plsc.VectorSubcoreMesh(core_axis_name="c", subcore_axis_name="s"),
        compiler_params=pltpu.CompilerParams(needs_layout_passes=False),
        out_type=[
            jax.ShapeDtypeStruct((NTILES, RPT, DHALF), f32),
            jax.ShapeDtypeStruct((NTILES, RPT, DHALF), f32),
        ],
        scratch_types=[
            pltpu.VMEM((NN,), f32),
            pltpu.VMEM((BATCH,), jnp.int32),
            pltpu.VMEM((BATCH,), jnp.int32),
            pltpu.VMEM((BATCH, DHALF), f32),
            pltpu.VMEM_SHARED((NN, DHALF), f32),
            pltpu.SemaphoreType.DMA,
        ],
    )
    return kern(_w_body)(src, dst, cnt)


def _tail_body(acc0_ref, acc1_ref, yr_ref, cnt_ref, w_ref,
               gamma_ref, beta_ref, wl2_ref, bl2_ref, wr2_ref,
               wsk_ref, bsk_ref, wc1_ref, bc1_ref, wc2_ref, bc2_ref,
               out_ref, sum_ref, sq_ref, s1_ref, m1_ref):
    p = pl.program_id(0)
    i = pl.program_id(1)

    @pl.when((p == 0) & (i == 0))
    def _():
        sum_ref[...] = jnp.zeros_like(sum_ref)

    @pl.when((p == 1) & (i == 0))
    def _():
        sq_ref[...] = jnp.zeros_like(sq_ref)

    @pl.when((p == 2) & (i == 0))
    def _():
        s1_ref[...] = jnp.zeros_like(s1_ref)
        m1_ref[...] = jnp.zeros_like(m1_ref)

    acc = jnp.concatenate([acc0_ref[...], acc1_ref[...]], axis=1)
    z = acc / jnp.maximum(cnt_ref[...], 1.0) + yr_ref[...]

    @pl.when(p == 0)
    def _():
        sum_ref[...] += jnp.sum(z, axis=0, keepdims=True)

    @pl.when(p == 1)
    def _():
        zc = z - sum_ref[...] * (1.0 / NN)
        sq_ref[...] += jnp.sum(zc * zc, axis=0, keepdims=True)

    @pl.when(p == 2)
    def _():
        mu = sum_ref[...] * (1.0 / NN)
        var = sq_ref[...] * (1.0 / NN)
        inv = 1.0 / jnp.sqrt(var + 1e-5)
        h = jnp.maximum((z - mu) * (inv * gamma_ref[...]) + beta_ref[...], 0.0)
        s1_ref[...] += jnp.sum(h * w_ref[...], axis=0, keepdims=True)
        m1_ref[...] += jnp.sum(h, axis=0, keepdims=True)

    @pl.when((p == 2) & (i == NN // DBLK - 1))
    def _():
        s1 = s1_ref[...] * (1.0 / NN)
        m1 = m1_ref[...] * (1.0 / NN)
        mh2 = (jnp.dot(s1, wl2_ref[...], preferred_element_type=jnp.float32, precision=_PHI)
               + bl2_ref[...]
               + jnp.dot(m1, wr2_ref[...], preferred_element_type=jnp.float32, precision=_PHI))
        pooled = jnp.dot(mh2, wsk_ref[...], preferred_element_type=jnp.float32, precision=_PHI) + bsk_ref[...]
        hid = jnp.maximum(
            jnp.dot(pooled, wc1_ref[...], preferred_element_type=jnp.float32, precision=_PHI) + bc1_ref[...], 0.0)
        out_ref[...] = jnp.dot(hid, wc2_ref[...], preferred_element_type=jnp.float32, precision=_PHI) + bc2_ref[...]


def _finalize(acc0, acc1, yr, cnt2, w2, gamma2, beta2, W_l2, b_l2_2, W_r2,
              W_sk, b_sk_2, W_c1, b_c1_2, W_c2, b_c2_2):
    def full(a):
        n = len(a.shape)
        return pl.BlockSpec(a.shape, lambda p, i, _n=n: (0,) * _n)
    return pl.pallas_call(
        _tail_body,
        grid=(3, NN // DBLK),
        in_specs=[
            pl.BlockSpec((DBLK, DHALF), lambda p, i: (i, 0)),
            pl.BlockSpec((DBLK, DHALF), lambda p, i: (i, 0)),
            pl.BlockSpec((DBLK, DIN), lambda p, i: (i, 0)),
            pl.BlockSpec((DBLK, 1), lambda p, i: (i, 0)),
            pl.BlockSpec((DBLK, 1), lambda p, i: (i, 0)),
            full(gamma2), full(beta2), full(W_l2), full(b_l2_2), full(W_r2),
            full(W_sk), full(b_sk_2), full(W_c1), full(b_c1_2), full(W_c2), full(b_c2_2),
        ],
        out_specs=pl.BlockSpec((1, 2), lambda p, i: (0, 0)),
        out_shape=jax.ShapeDtypeStruct((1, 2), jnp.float32),
        scratch_shapes=[
            pltpu.VMEM((1, DIN), jnp.float32),
            pltpu.VMEM((1, DIN), jnp.float32),
            pltpu.VMEM((1, DIN), jnp.float32),
            pltpu.VMEM((1, DIN), jnp.float32),
        ],
    )(acc0, acc1, yr, cnt2, w2, gamma2, beta2, W_l2, b_l2_2, W_r2,
      W_sk, b_sk_2, W_c1, b_c1_2, W_c2, b_c2_2)


def kernel(x, edge_index, W_l1, b_l1, W_r1, gamma, beta, W_l2, b_l2, W_r2,
           W_sk, b_sk, W_c1, b_c1, W_c2, b_c2):
    src = edge_index[0]
    dst = edge_index[1]

    y_cat, y_r = _matmul_stage(x, W_l1, W_r1, b_l1.reshape(1, -1))
    acc0, acc1 = _aggregate(src, dst, y_cat.reshape(2 * NN, DHALF))
    acc0 = acc0.reshape(NN, DHALF)
    acc1 = acc1.reshape(NN, DHALF)
    cnt0, cnt1 = _degree_counts(dst)
    cnt = cnt0.reshape(NN, DHALF)[:, 0] + cnt1.reshape(NN, DHALF)[:, 0]
    w0, w1 = _edge_weights(src, dst, cnt)
    w = w0.reshape(NN, DHALF)[:, 0] + w1.reshape(NN, DHALF)[:, 0]

    return _finalize(
        acc0, acc1, y_r, cnt[:, None], w[:, None],
        gamma.reshape(1, -1), beta.reshape(1, -1), W_l2, b_l2.reshape(1, -1),
        W_r2, W_sk, b_sk.reshape(1, -1), W_c1, b_c1.reshape(1, -1),
        W_c2, b_c2.reshape(1, -1))


# speedup vs baseline: 9.0620x; 1.2648x over previous
import functools

import jax
import jax.numpy as jnp
from jax import lax
from jax.experimental import pallas as pl
from jax.experimental.pallas import tpu as pltpu
from jax.experimental.pallas import tpu_sc as plsc

_PHI = jax.lax.Precision.HIGHEST

NN = 10000
EE = 160000
DIN = 256
DHALF = 128
BATCH = 128
NBATCH = EE // BATCH
NTILES = 16
RPT = NN // NTILES
HROWS = 80
MMBLK = 2000
DBLK = 1000


def _mm_body(x_ref, wl_ref, wr_ref, bl_ref, ycat_ref, yr_ref):
    xx = x_ref[...]
    yl = jnp.dot(xx, wl_ref[...], preferred_element_type=jnp.float32, precision=_PHI)
    ycat_ref[0] = yl[:, :DHALF]
    ycat_ref[1] = yl[:, DHALF:]
    yr_ref[...] = jnp.dot(xx, wr_ref[...], preferred_element_type=jnp.float32, precision=_PHI) + bl_ref[...]


def _matmul_stage(x, W_l1, W_r1, b_l1_2d):
    return pl.pallas_call(
        _mm_body,
        grid=(NN // MMBLK,),
        in_specs=[
            pl.BlockSpec((MMBLK, DIN), lambda i: (i, 0)),
            pl.BlockSpec((DIN, DIN), lambda i: (0, 0)),
            pl.BlockSpec((DIN, DIN), lambda i: (0, 0)),
            pl.BlockSpec((1, DIN), lambda i: (0, 0)),
        ],
        out_specs=[
            pl.BlockSpec((2, MMBLK, DHALF), lambda i: (0, i, 0)),
            pl.BlockSpec((MMBLK, DIN), lambda i: (i, 0)),
        ],
        out_shape=[
            jax.ShapeDtypeStruct((2, NN, DHALF), jnp.float32),
            jax.ShapeDtypeStruct((NN, DIN), jnp.float32),
        ],
    )(x, W_l1, W_r1, b_l1_2d)


def _zero_rows(buf, nrows, width):
    def _row(i, _):
        for jj in range(width // 16):
            buf[i, pl.ds(jj * 16, 16)] = jnp.zeros((16,), jnp.float32)
        return 0
    lax.fori_loop(0, nrows, _row, 0)


def _zero_spmem(sh, zsrc, s, width):
    nzc = NN // 128
    nz = jnp.where(s < nzc - (nzc // NTILES) * NTILES,
                   nzc // NTILES + 1, nzc // NTILES)

    def _zchunk(k, _):
        m = (s + k * NTILES) * 128
        pltpu.sync_copy(zsrc, sh.at[pl.ds(m, 128)])
        return 0
    lax.fori_loop(0, nz, _zchunk, 0)

    @pl.when(s == NTILES - 1)
    def _():
        pltpu.sync_copy(zsrc.at[pl.ds(0, 16)], sh.at[pl.ds(nzc * 128, 16)])


def _agg_body(src_hbm, dst_hbm, ycat_hbm, acc0_out, acc1_out, cnt0_out, cnt1_out,
              src_a, dst_a, rows_a, src_b, dst_b, rows_b, hist, iota_h,
              acc_sh, cnt_sh, sem_a, sem_b):
    c = lax.axis_index("c")
    s = lax.axis_index("s")
    row0 = s * RPT

    _zero_rows(hist, HROWS, DHALF)
    for i in range(HROWS // 16):
        iota_h[pl.ds(i * 16, 16)] = lax.iota(jnp.int32, 16) + i * 16

    _zero_rows(rows_a, BATCH, DHALF)
    _zero_spmem(acc_sh, rows_a, s, DHALF)

    @pl.when(s == 0)
    def _():
        pltpu.sync_copy(rows_a.at[pl.ds(0, HROWS)], cnt_sh)
    plsc.subcore_barrier()

    nb = jnp.where(s < NBATCH - (NBATCH // NTILES) * NTILES,
                   NBATCH // NTILES + 1, NBATCH // NTILES)
    coff = c * NN

    def _start(j, src_v, dst_v, rows_v, sem):
        off = (s + j * NTILES) * BATCH
        pltpu.sync_copy(src_hbm.at[pl.ds(off, BATCH)], src_v)
        pltpu.sync_copy(dst_hbm.at[pl.ds(off, BATCH)], dst_v)
        for g in range(BATCH // 16):
            src_v[pl.ds(g * 16, 16)] = src_v[pl.ds(g * 16, 16)] + coff
        return pltpu.async_copy(ycat_hbm.at[src_v], rows_v, sem)

    ones16 = jnp.ones((16,), jnp.float32)
    count_mine = (s % 2) == c

    def _drain(src_v, dst_v, rows_v, sem):
        pltpu.make_async_copy(ycat_hbm.at[src_v], rows_v, sem).wait()
        pltpu.sync_copy(rows_v, acc_sh.at[dst_v], add=True)

        @pl.when(count_mine)
        def _():
            for g in range(BATCH // 16):
                dv = dst_v[pl.ds(g * 16, 16)]
                plsc.addupdate_scatter(
                    hist, [lax.shift_right_logical(dv, 7), dv & 127], ones16)

    _start(0, src_a, dst_a, rows_a, sem_a)
    npairs = (nb + 1) // 2

    def _pair(m, _):
        j1 = 2 * m + 1

        @pl.when(j1 < nb)
        def _():
            _start(j1, src_b, dst_b, rows_b, sem_b)
        _drain(src_a, dst_a, rows_a, sem_a)

        @pl.when(j1 < nb)
        def _():
            @pl.when(j1 + 1 < nb)
            def _():
                _start(j1 + 1, src_a, dst_a, rows_a, sem_a)
            _drain(src_b, dst_b, rows_b, sem_b)
        return 0

    lax.fori_loop(0, npairs, _pair, 0)

    @pl.when(count_mine)
    def _():
        pltpu.sync_copy(hist, cnt_sh.at[iota_h], add=True)
    plsc.subcore_barrier()

    @pl.when(c == 0)
    def _():
        pltpu.sync_copy(acc_sh.at[pl.ds(row0, RPT)], acc0_out.at[s])

    @pl.when(c == 1)
    def _():
        pltpu.sync_copy(acc_sh.at[pl.ds(row0, RPT)], acc1_out.at[s])

    @pl.when((s == 0) & (c == 0))
    def _():
        pltpu.sync_copy(cnt_sh, cnt0_out)

    @pl.when((s == 0) & (c == 1))
    def _():
        pltpu.sync_copy(cnt_sh, cnt1_out)


def _aggregate(src, dst, y_cat_flat):
    f32 = jnp.float32
    kern = functools.partial(
        pl.kernel,
        mesh=plsc.VectorSubcoreMesh(core_axis_name="c", subcore_axis_name="s"),
        compiler_params=pltpu.CompilerParams(needs_layout_passes=False),
        out_type=[
            jax.ShapeDtypeStruct((NTILES, RPT, DHALF), f32),
            jax.ShapeDtypeStruct((NTILES, RPT, DHALF), f32),
            jax.ShapeDtypeStruct((HROWS, DHALF), f32),
            jax.ShapeDtypeStruct((HROWS, DHALF), f32),
        ],
        scratch_types=[
            pltpu.VMEM((BATCH,), jnp.int32),
            pltpu.VMEM((BATCH,), jnp.int32),
            pltpu.VMEM((BATCH, DHALF), f32),
            pltpu.VMEM((BATCH,), jnp.int32),
            pltpu.VMEM((BATCH,), jnp.int32),
            pltpu.VMEM((BATCH, DHALF), f32),
            pltpu.VMEM((HROWS, DHALF), f32),
            pltpu.VMEM((HROWS,), jnp.int32),
            pltpu.VMEM_SHARED((NN, DHALF), f32),
            pltpu.VMEM_SHARED((HROWS, DHALF), f32),
            pltpu.SemaphoreType.DMA,
            pltpu.SemaphoreType.DMA,
        ],
    )
    return kern(_agg_body)(src, dst, y_cat_flat)


def _w_body(src_hbm, dst_hbm, cnt_hbm, w0_out, w1_out,
            cnt_vmem, src_slab, dst_slab, hist, iota_h, w_sh, sem):
    c = lax.axis_index("c")
    s = lax.axis_index("s")
    wid = s * 2 + c

    pltpu.sync_copy(cnt_hbm, cnt_vmem)

    _zero_rows(hist, HROWS, DHALF)
    for i in range(HROWS // 16):
        iota_h[pl.ds(i * 16, 16)] = lax.iota(jnp.int32, 16) + i * 16

    @pl.when((s == 0) & True)
    def _():
        pltpu.sync_copy(hist, w_sh)
    plsc.subcore_barrier()

    nw = NTILES * 2
    nb = jnp.where(wid < NBATCH - (NBATCH // nw) * nw,
                   NBATCH // nw + 1, NBATCH // nw)

    def _batch(k, _):
        off = (wid + k * nw) * BATCH
        pltpu.sync_copy(src_hbm.at[pl.ds(off, BATCH)], src_slab)
        pltpu.sync_copy(dst_hbm.at[pl.ds(off, BATCH)], dst_slab)
        for g in range(BATCH // 16):
            dstv = dst_slab[pl.ds(g * 16, 16)]
            cv = plsc.load_gather(cnt_vmem, [dstv])
            r = 1.0 / jnp.maximum(cv, 1.0)
            sv = src_slab[pl.ds(g * 16, 16)]
            plsc.addupdate_scatter(
                hist, [lax.shift_right_logical(sv, 7), sv & 127], r)
        return 0

    lax.fori_loop(0, nb, _batch, 0)

    pltpu.sync_copy(hist, w_sh.at[iota_h], add=True)
    plsc.subcore_barrier()

    @pl.when((s == 0) & (c == 0))
    def _():
        pltpu.sync_copy(w_sh, w0_out)

    @pl.when((s == 0) & (c == 1))
    def _():
        pltpu.sync_copy(w_sh, w1_out)


def _edge_weights(src, dst, cnt):
    f32 = jnp.float32
    kern = functools.partial(
        pl.kernel,
        mesh=plsc.VectorSubcoreMesh(core_axis_name="c", subcore_axis_name="s"),
        compiler_params=pltpu.CompilerParams(needs_layout_passes=False),
        out_type=[
            jax.ShapeDtypeStruct((HROWS, DHALF), f32),
            jax.ShapeDtypeStruct((HROWS, DHALF), f32),
        ],
        scratch_types=[
            pltpu.VMEM((NN,), f32),
            pltpu.VMEM((BATCH,), jnp.int32),
            pltpu.VMEM((BATCH,), jnp.int32),
            pltpu.VMEM((HROWS, DHALF), f32),
            pltpu.VMEM((HROWS,), jnp.int32),
            pltpu.VMEM_SHARED((HROWS, DHALF), f32),
            pltpu.SemaphoreType.DMA,
        ],
    )
    return kern(_w_body)(src, dst, cnt)


def _tail_body(acc0_ref, acc1_ref, yr_ref, cnt_ref, w_ref,
               gamma_ref, beta_ref, wl2_ref, bl2_ref, wr2_ref,
               wsk_ref, bsk_ref, wc1_ref, bc1_ref, wc2_ref, bc2_ref,
               out_ref, sum_ref, sq_ref, s1_ref, m1_ref):
    p = pl.program_id(0)
    i = pl.program_id(1)

    @pl.when((p == 0) & (i == 0))
    def _():
        sum_ref[...] = jnp.zeros_like(sum_ref)

    @pl.when((p == 1) & (i == 0))
    def _():
        sq_ref[...] = jnp.zeros_like(sq_ref)

    @pl.when((p == 2) & (i == 0))
    def _():
        s1_ref[...] = jnp.zeros_like(s1_ref)
        m1_ref[...] = jnp.zeros_like(m1_ref)

    acc = jnp.concatenate([acc0_ref[...], acc1_ref[...]], axis=1)
    z = acc / jnp.maximum(cnt_ref[...], 1.0) + yr_ref[...]

    @pl.when(p == 0)
    def _():
        sum_ref[...] += jnp.sum(z, axis=0, keepdims=True)

    @pl.when(p == 1)
    def _():
        zc = z - sum_ref[...] * (1.0 / NN)
        sq_ref[...] += jnp.sum(zc * zc, axis=0, keepdims=True)

    @pl.when(p == 2)
    def _():
        mu = sum_ref[...] * (1.0 / NN)
        var = sq_ref[...] * (1.0 / NN)
        inv = 1.0 / jnp.sqrt(var + 1e-5)
        h = jnp.maximum((z - mu) * (inv * gamma_ref[...]) + beta_ref[...], 0.0)
        s1_ref[...] += jnp.sum(h * w_ref[...], axis=0, keepdims=True)
        m1_ref[...] += jnp.sum(h, axis=0, keepdims=True)

    @pl.when((p == 2) & (i == NN // DBLK - 1))
    def _():
        s1 = s1_ref[...] * (1.0 / NN)
        m1 = m1_ref[...] * (1.0 / NN)
        mh2 = (jnp.dot(s1, wl2_ref[...], preferred_element_type=jnp.float32, precision=_PHI)
               + bl2_ref[...]
               + jnp.dot(m1, wr2_ref[...], preferred_element_type=jnp.float32, precision=_PHI))
        pooled = jnp.dot(mh2, wsk_ref[...], preferred_element_type=jnp.float32, precision=_PHI) + bsk_ref[...]
        hid = jnp.maximum(
            jnp.dot(pooled, wc1_ref[...], preferred_element_type=jnp.float32, precision=_PHI) + bc1_ref[...], 0.0)
        out_ref[...] = jnp.dot(hid, wc2_ref[...], preferred_element_type=jnp.float32, precision=_PHI) + bc2_ref[...]


def _finalize(acc0, acc1, yr, cnt2, w2, gamma2, beta2, W_l2, b_l2_2, W_r2,
              W_sk, b_sk_2, W_c1, b_c1_2, W_c2, b_c2_2):
    def full(a):
        n = len(a.shape)
        return pl.BlockSpec(a.shape, lambda p, i, _n=n: (0,) * _n)
    return pl.pallas_call(
        _tail_body,
        grid=(3, NN // DBLK),
        in_specs=[
            pl.BlockSpec((DBLK, DHALF), lambda p, i: (i, 0)),
            pl.BlockSpec((DBLK, DHALF), lambda p, i: (i, 0)),
            pl.BlockSpec((DBLK, DIN), lambda p, i: (i, 0)),
            pl.BlockSpec((DBLK, 1), lambda p, i: (i, 0)),
            pl.BlockSpec((DBLK, 1), lambda p, i: (i, 0)),
            full(gamma2), full(beta2), full(W_l2), full(b_l2_2), full(W_r2),
            full(W_sk), full(b_sk_2), full(W_c1), full(b_c1_2), full(W_c2), full(b_c2_2),
        ],
        out_specs=pl.BlockSpec((1, 2), lambda p, i: (0, 0)),
        out_shape=jax.ShapeDtypeStruct((1, 2), jnp.float32),
        scratch_shapes=[
            pltpu.VMEM((1, DIN), jnp.float32),
            pltpu.VMEM((1, DIN), jnp.float32),
            pltpu.VMEM((1, DIN), jnp.float32),
            pltpu.VMEM((1, DIN), jnp.float32),
        ],
    )(acc0, acc1, yr, cnt2, w2, gamma2, beta2, W_l2, b_l2_2, W_r2,
      W_sk, b_sk_2, W_c1, b_c1_2, W_c2, b_c2_2)


def kernel(x, edge_index, W_l1, b_l1, W_r1, gamma, beta, W_l2, b_l2, W_r2,
           W_sk, b_sk, W_c1, b_c1, W_c2, b_c2):
    src = edge_index[0]
    dst = edge_index[1]

    y_cat, y_r = _matmul_stage(x, W_l1, W_r1, b_l1.reshape(1, -1))
    acc0, acc1, cnt0, cnt1 = _aggregate(src, dst, y_cat.reshape(2 * NN, DHALF))
    acc0 = acc0.reshape(NN, DHALF)
    acc1 = acc1.reshape(NN, DHALF)
    cnt = (cnt0 + cnt1).reshape(HROWS * DHALF)[:NN]
    w0, w1 = _edge_weights(src, dst, cnt)
    w = (w0 + w1).reshape(HROWS * DHALF)[:NN]

    return _finalize(
        acc0, acc1, y_r, cnt[:, None], w[:, None],
        gamma.reshape(1, -1), beta.reshape(1, -1), W_l2, b_l2.reshape(1, -1),
        W_r2, W_sk, b_sk.reshape(1, -1), W_c1, b_c1.reshape(1, -1),
        W_c2, b_c2.reshape(1, -1))
